# Initial kernel scaffold; baseline (speedup 1.0000x reference)
import jax
import jax.numpy as jnp
from jax.experimental import pallas as pl


def kernel(mem, ent_mem, val, prob, idx):
    raise NotImplementedError("write your pallas kernel here")


# speedup vs baseline: 1.6100x; 1.6100x over previous
import functools

import jax
import jax.numpy as jnp
from jax import lax
from jax.experimental import pallas as pl
from jax.experimental.pallas import tpu as pltpu
from jax.experimental.pallas import tpu_sc as plsc

_BETA = 5.5
_M, _D, _B, _C = 65536, 512, 16384, 1000
_L = 16
_NC, _NS = 2, 16
_NW = _NC * _NS
_SPT = _M // _NW
_WCH = _SPT // _L
_SCH = _B // _L
_JBITS = 14
_ROWS_BLOCK = 256
_I32MAX = jnp.iinfo(jnp.int32).max


def _stats_body(prob_ref, val_ref, ent_ref, valn_ref):
    x = prob_ref[...] * _BETA
    m = jnp.max(x, axis=-1, keepdims=True)
    e = jnp.exp(x - m)
    z = jnp.sum(e, axis=-1, keepdims=True)
    p = e / z
    ent_ref[0, 0, :] = -jnp.sum(p * jnp.log(p + 1e-10), axis=-1)
    v = val_ref[...]
    n = jnp.sqrt(jnp.sum(v * v, axis=-1, keepdims=True))
    valn_ref[...] = v / n


_NB = _B // _ROWS_BLOCK

_stats_call = pl.pallas_call(
    _stats_body,
    grid=(_NB,),
    in_specs=[
        pl.BlockSpec((_ROWS_BLOCK, _C), lambda i: (i, 0)),
        pl.BlockSpec((_ROWS_BLOCK, _D), lambda i: (i, 0)),
    ],
    out_specs=[
        pl.BlockSpec((1, 1, _ROWS_BLOCK), lambda i: (i, 0, 0)),
        pl.BlockSpec((_ROWS_BLOCK, _D), lambda i: (i, 0)),
    ],
    out_shape=[
        jax.ShapeDtypeStruct((_NB, 1, _ROWS_BLOCK), jnp.float32),
        jax.ShapeDtypeStruct((_B, _D), jnp.float32),
    ],
)


def _gather16(x, idx):
    return lax.gather(
        x,
        idx[:, None],
        lax.GatherDimensionNumbers(
            offset_dims=(), collapsed_slice_dims=(0,), start_index_map=(0,)
        ),
        slice_sizes=(1,),
        mode=lax.GatherScatterMode.PROMISE_IN_BOUNDS,
    )


_RCH = 64
_NCP = _SPT // _RCH


def _sc_body(mem_hbm, entmem_hbm, idx_hbm, ent_hbm, valn_hbm, out_hbm,
             idx_v, ent_v, entmem_v, lastw_v, slist_v, jlist_v, rows_v,
             cbuf_v, copy_sem, dma_sem):
    wid = lax.axis_index("s") * _NC + lax.axis_index("c")
    base = wid * _SPT

    stores = []
    for c in range(_NCP):
        b = c % 2
        if c >= 2:
            stores[c - 2].wait()
        pltpu.async_copy(
            mem_hbm.at[pl.ds(base + c * _RCH, _RCH)], cbuf_v.at[b], dma_sem
        ).wait()
        stores.append(pltpu.async_copy(
            cbuf_v.at[b], out_hbm.at[pl.ds(base + c * _RCH, _RCH)], copy_sem
        ))

    pltpu.sync_copy(idx_hbm, idx_v)
    pltpu.sync_copy(ent_hbm, ent_v)
    pltpu.sync_copy(entmem_hbm.at[pl.ds(base, _SPT)], entmem_v)

    iota = lax.iota(jnp.int32, _L)
    neg1 = jnp.full((_L,), -1, jnp.int32)
    big = jnp.full((_L,), _I32MAX, jnp.int32)
    nxt = jnp.minimum(iota + 1, _L - 1)
    zero16 = jnp.zeros((_L,), jnp.int32)

    def init_body(k, c):
        lastw_v[pl.ds(k * _L, _L)] = neg1
        return c

    lax.fori_loop(0, _WCH, init_body, 0)

    def scan_body(i, c):
        sv = idx_v[pl.ds(i * _L, _L)]
        jv = i * _L + iota
        inr = (sv >= base) & (sv < base + _SPT)
        comb = jnp.where(inr, (sv << _JBITS) + jv, big)
        srt = plsc.sort_key_val(comb, comb)[0]
        s_s = lax.shift_right_arithmetic(srt, _JBITS)
        j_s = lax.bitwise_and(srt, _B - 1)
        valid = srt != big
        s_nxt = _gather16(s_s, nxt)
        lastrun = (s_s != s_nxt) | (iota == _L - 1)
        plsc.store_scatter(lastw_v, [s_s - base], j_s, mask=valid & lastrun)
        return c

    lax.fori_loop(0, _SCH, scan_body, 0)

    def win_body(k, cnt):
        wj = lastw_v[pl.ds(k * _L, _L)]
        written = wj >= 0
        entw = plsc.load_gather(ent_v, [wj], mask=written)
        sent = entmem_v[pl.ds(k * _L, _L)]
        win = written & (entw < sent)
        wi = win.astype(jnp.int32)
        pos = cnt + plsc.cumsum(wi) - wi
        svec = base + k * _L + iota
        plsc.store_scatter(slist_v, [pos], svec, mask=win)
        plsc.store_scatter(jlist_v, [pos], wj, mask=win)
        return cnt + jnp.sum(wi)

    cnt = lax.fori_loop(0, _WCH, win_body, jnp.int32(0))

    stores[-2].wait()
    stores[-1].wait()

    nch = (cnt + _L - 1) // _L

    def dma_body(c, carry):
        lanemask = (c * _L + iota) < cnt
        sv = slist_v[pl.ds(c * _L, _L)]
        jv = jlist_v[pl.ds(c * _L, _L)]
        sv = jnp.where(lanemask, sv, _gather16(sv, zero16))
        jv = jnp.where(lanemask, jv, _gather16(jv, zero16))
        pltpu.async_copy(valn_hbm.at[jv], rows_v, dma_sem).wait()
        pltpu.async_copy(rows_v, out_hbm.at[sv], dma_sem).wait()
        return carry

    lax.fori_loop(0, nch, dma_body, 0)


_sc_call = pl.kernel(
    _sc_body,
    out_type=jax.ShapeDtypeStruct((_M, _D), jnp.float32),
    mesh=plsc.VectorSubcoreMesh(
        core_axis_name="c", subcore_axis_name="s", num_cores=_NC,
        num_subcores=_NS,
    ),
    scratch_types=[
        pltpu.VMEM((_B,), jnp.int32),
        pltpu.VMEM((_B,), jnp.float32),
        pltpu.VMEM((_SPT,), jnp.float32),
        pltpu.VMEM((_SPT,), jnp.int32),
        pltpu.VMEM((_SPT,), jnp.int32),
        pltpu.VMEM((_SPT,), jnp.int32),
        pltpu.VMEM((_L, _D), jnp.float32),
        pltpu.VMEM((2, _RCH, _D), jnp.float32),
        pltpu.SemaphoreType.DMA,
        pltpu.SemaphoreType.DMA,
    ],
    compiler_params=pltpu.CompilerParams(needs_layout_passes=False),
)


def kernel(mem, ent_mem, val, prob, idx):
    ent3, val_n = _stats_call(prob, val)
    ent = ent3.reshape(_B)
    return _sc_call(mem, ent_mem, idx.astype(jnp.int32), ent, val_n)


# speedup vs baseline: 1.9810x; 1.2305x over previous
import functools

import jax
import jax.numpy as jnp
from jax import lax
from jax.experimental import pallas as pl
from jax.experimental.pallas import tpu as pltpu
from jax.experimental.pallas import tpu_sc as plsc

_BETA = 5.5
_M, _D, _B, _C = 65536, 512, 16384, 1000
_L = 16
_NC, _NS = 2, 16
_NW = _NC * _NS
_SPT = _M // _NW
_WCH = _SPT // _L
_SCH = _B // _L
_JBITS = 14
_ROWS_BLOCK = 256
_I32MAX = jnp.iinfo(jnp.int32).max


def _stats_body(probt_ref, val_ref, ent_ref, valn_ref):
    x = probt_ref[...] * _BETA
    m = jnp.max(x, axis=0, keepdims=True)
    e = jnp.exp(x - m)
    z = jnp.sum(e, axis=0, keepdims=True)
    p = e / z
    ent_ref[0, 0, :] = -jnp.sum(p * jnp.log(p + 1e-10), axis=0)
    v = val_ref[...]
    n = jnp.sqrt(jnp.sum(v * v, axis=-1, keepdims=True))
    valn_ref[...] = v / n


_NB = _B // _ROWS_BLOCK

_stats_call = pl.pallas_call(
    _stats_body,
    grid=(_NB,),
    in_specs=[
        pl.BlockSpec((_C, _ROWS_BLOCK), lambda i: (0, i)),
        pl.BlockSpec((_ROWS_BLOCK, _D), lambda i: (i, 0)),
    ],
    out_specs=[
        pl.BlockSpec((1, 1, _ROWS_BLOCK), lambda i: (i, 0, 0)),
        pl.BlockSpec((_ROWS_BLOCK, _D), lambda i: (i, 0)),
    ],
    out_shape=[
        jax.ShapeDtypeStruct((_NB, 1, _ROWS_BLOCK), jnp.float32),
        jax.ShapeDtypeStruct((_B, _D), jnp.float32),
    ],
)


def _gather16(x, idx):
    return lax.gather(
        x,
        idx[:, None],
        lax.GatherDimensionNumbers(
            offset_dims=(), collapsed_slice_dims=(0,), start_index_map=(0,)
        ),
        slice_sizes=(1,),
        mode=lax.GatherScatterMode.PROMISE_IN_BOUNDS,
    )


_RCH = 64
_NCP = _SPT // _RCH


def _sc_body(mem_hbm, entmem_hbm, idx_hbm, ent_hbm, valn_hbm, out_hbm,
             idx_v, ent_v, entmem_v, lastw_v, slist_v, jlist_v, rows_v,
             cbuf_v, copy_sem, dma_sem):
    wid = lax.axis_index("s") * _NC + lax.axis_index("c")
    base = wid * _SPT

    stores = []
    for c in range(_NCP):
        b = c % 2
        if c >= 2:
            stores[c - 2].wait()
        pltpu.async_copy(
            mem_hbm.at[pl.ds(base + c * _RCH, _RCH)], cbuf_v.at[b], dma_sem
        ).wait()
        stores.append(pltpu.async_copy(
            cbuf_v.at[b], out_hbm.at[pl.ds(base + c * _RCH, _RCH)], copy_sem
        ))

    pltpu.sync_copy(idx_hbm, idx_v)
    pltpu.sync_copy(ent_hbm, ent_v)
    pltpu.sync_copy(entmem_hbm.at[pl.ds(base, _SPT)], entmem_v)

    iota = lax.iota(jnp.int32, _L)
    neg1 = jnp.full((_L,), -1, jnp.int32)
    big = jnp.full((_L,), _I32MAX, jnp.int32)
    nxt = jnp.minimum(iota + 1, _L - 1)
    zero16 = jnp.zeros((_L,), jnp.int32)

    def init_body(k, c):
        lastw_v[pl.ds(k * _L, _L)] = neg1
        return c

    lax.fori_loop(0, _WCH, init_body, 0)

    def scan_body(i, c):
        sv = idx_v[pl.ds(i * _L, _L)]
        jv = i * _L + iota
        inr = (sv >= base) & (sv < base + _SPT)
        comb = jnp.where(inr, (sv << _JBITS) + jv, big)
        srt = plsc.sort_key_val(comb, comb)[0]
        s_s = lax.shift_right_arithmetic(srt, _JBITS)
        j_s = lax.bitwise_and(srt, _B - 1)
        valid = srt != big
        s_nxt = _gather16(s_s, nxt)
        lastrun = (s_s != s_nxt) | (iota == _L - 1)
        plsc.store_scatter(lastw_v, [s_s - base], j_s, mask=valid & lastrun)
        return c

    lax.fori_loop(0, _SCH, scan_body, 0)

    def win_body(k, cnt):
        wj = lastw_v[pl.ds(k * _L, _L)]
        written = wj >= 0
        entw = plsc.load_gather(ent_v, [wj], mask=written)
        sent = entmem_v[pl.ds(k * _L, _L)]
        win = written & (entw < sent)
        wi = win.astype(jnp.int32)
        pos = cnt + plsc.cumsum(wi) - wi
        svec = base + k * _L + iota
        plsc.store_scatter(slist_v, [pos], svec, mask=win)
        plsc.store_scatter(jlist_v, [pos], wj, mask=win)
        return cnt + jnp.sum(wi)

    cnt = lax.fori_loop(0, _WCH, win_body, jnp.int32(0))

    stores[-2].wait()
    stores[-1].wait()

    nch = (cnt + _L - 1) // _L

    def dma_body(c, carry):
        lanemask = (c * _L + iota) < cnt
        sv = slist_v[pl.ds(c * _L, _L)]
        jv = jlist_v[pl.ds(c * _L, _L)]
        sv = jnp.where(lanemask, sv, _gather16(sv, zero16))
        jv = jnp.where(lanemask, jv, _gather16(jv, zero16))
        pltpu.async_copy(valn_hbm.at[jv], rows_v, dma_sem).wait()
        pltpu.async_copy(rows_v, out_hbm.at[sv], dma_sem).wait()
        return carry

    lax.fori_loop(0, nch, dma_body, 0)


_sc_call = pl.kernel(
    _sc_body,
    out_type=jax.ShapeDtypeStruct((_M, _D), jnp.float32),
    mesh=plsc.VectorSubcoreMesh(
        core_axis_name="c", subcore_axis_name="s", num_cores=_NC,
        num_subcores=_NS,
    ),
    scratch_types=[
        pltpu.VMEM((_B,), jnp.int32),
        pltpu.VMEM((_B,), jnp.float32),
        pltpu.VMEM((_SPT,), jnp.float32),
        pltpu.VMEM((_SPT,), jnp.int32),
        pltpu.VMEM((_SPT,), jnp.int32),
        pltpu.VMEM((_SPT,), jnp.int32),
        pltpu.VMEM((_L, _D), jnp.float32),
        pltpu.VMEM((2, _RCH, _D), jnp.float32),
        pltpu.SemaphoreType.DMA,
        pltpu.SemaphoreType.DMA,
    ],
    compiler_params=pltpu.CompilerParams(needs_layout_passes=False),
)


def kernel(mem, ent_mem, val, prob, idx):
    ent3, val_n = _stats_call(prob.T, val)
    ent = ent3.reshape(_B)
    return _sc_call(mem, ent_mem, idx.astype(jnp.int32), ent, val_n)


# speedup vs baseline: 2.2457x; 1.1336x over previous
import functools

import jax
import jax.numpy as jnp
from jax import lax
from jax.experimental import pallas as pl
from jax.experimental.pallas import tpu as pltpu
from jax.experimental.pallas import tpu_sc as plsc

_BETA = 5.5
_M, _D, _B, _C = 65536, 512, 16384, 1000
_L = 16
_NC, _NS = 2, 16
_NW = _NC * _NS
_SPT = _M // _NW
_WCH = _SPT // _L
_SCH = _B // _L
_JBITS = 14
_ROWS_BLOCK = 256
_I32MAX = jnp.iinfo(jnp.int32).max


def _stats_body(probt_ref, val_ref, ent_ref, valn_ref):
    x = probt_ref[...] * _BETA
    m = jnp.max(x, axis=0, keepdims=True)
    xt = x - m
    e = jnp.exp(xt)
    z = jnp.sum(e, axis=0, keepdims=True)
    s = jnp.sum(e * xt, axis=0, keepdims=True)
    ent_ref[0, 0, :] = (jnp.log(z) - s / z)[0]
    v = val_ref[...]
    n = jnp.sqrt(jnp.sum(v * v, axis=-1, keepdims=True))
    valn_ref[...] = v / n


_NB = _B // _ROWS_BLOCK

_stats_call = pl.pallas_call(
    _stats_body,
    grid=(_NB,),
    in_specs=[
        pl.BlockSpec((_C, _ROWS_BLOCK), lambda i: (0, i)),
        pl.BlockSpec((_ROWS_BLOCK, _D), lambda i: (i, 0)),
    ],
    out_specs=[
        pl.BlockSpec((1, 1, _ROWS_BLOCK), lambda i: (i, 0, 0)),
        pl.BlockSpec((_ROWS_BLOCK, _D), lambda i: (i, 0)),
    ],
    out_shape=[
        jax.ShapeDtypeStruct((_NB, 1, _ROWS_BLOCK), jnp.float32),
        jax.ShapeDtypeStruct((_B, _D), jnp.float32),
    ],
)


def _gather16(x, idx):
    return lax.gather(
        x,
        idx[:, None],
        lax.GatherDimensionNumbers(
            offset_dims=(), collapsed_slice_dims=(0,), start_index_map=(0,)
        ),
        slice_sizes=(1,),
        mode=lax.GatherScatterMode.PROMISE_IN_BOUNDS,
    )


_RCH = 64
_NCP = _SPT // _RCH


def _sc_body(mem_hbm, entmem_hbm, idx_hbm, ent_hbm, valn_hbm, out_hbm,
             idx_v, ent_v, entmem_v, lastw_v, slist_v, jlist_v, rows_v,
             cbuf_v, copy_sem, dma_sem):
    wid = lax.axis_index("s") * _NC + lax.axis_index("c")
    base = wid * _SPT

    pltpu.sync_copy(idx_hbm, idx_v)
    pltpu.sync_copy(ent_hbm, ent_v)
    pltpu.sync_copy(entmem_hbm.at[pl.ds(base, _SPT)], entmem_v)

    iota = lax.iota(jnp.int32, _L)
    neg1 = jnp.full((_L,), -1, jnp.int32)
    big = jnp.full((_L,), _I32MAX, jnp.int32)
    nxt = jnp.minimum(iota + 1, _L - 1)
    zero16 = jnp.zeros((_L,), jnp.int32)

    def init_body(k, c):
        lastw_v[pl.ds(k * _L, _L)] = neg1
        return c

    lax.fori_loop(0, _WCH, init_body, 0)

    def scan_body(i, c):
        sv = idx_v[pl.ds(i * _L, _L)]
        jv = i * _L + iota
        inr = (sv >= base) & (sv < base + _SPT)
        comb = jnp.where(inr, (sv << _JBITS) + jv, big)
        srt = plsc.sort_key_val(comb, comb)[0]
        s_s = lax.shift_right_arithmetic(srt, _JBITS)
        j_s = lax.bitwise_and(srt, _B - 1)
        valid = srt != big
        s_nxt = _gather16(s_s, nxt)
        lastrun = (s_s != s_nxt) | (iota == _L - 1)
        plsc.store_scatter(lastw_v, [s_s - base], j_s, mask=valid & lastrun)
        return c

    _SEG = _SCH // _NCP
    stores = [None] * _NCP
    gathers = [None] * _NCP
    gathers[0] = pltpu.async_copy(
        mem_hbm.at[pl.ds(base, _RCH)], cbuf_v.at[0], dma_sem
    )
    for c in range(_NCP):
        gathers[c].wait()
        stores[c] = pltpu.async_copy(
            cbuf_v.at[c % 2], out_hbm.at[pl.ds(base + c * _RCH, _RCH)],
            copy_sem,
        )
        if c + 1 < _NCP:
            if c >= 1:
                stores[c - 1].wait()
            gathers[c + 1] = pltpu.async_copy(
                mem_hbm.at[pl.ds(base + (c + 1) * _RCH, _RCH)],
                cbuf_v.at[(c + 1) % 2], dma_sem,
            )
        lax.fori_loop(c * _SEG, (c + 1) * _SEG, scan_body, 0)

    def win_body(k, cnt):
        wj = lastw_v[pl.ds(k * _L, _L)]
        written = wj >= 0
        entw = plsc.load_gather(ent_v, [wj], mask=written)
        sent = entmem_v[pl.ds(k * _L, _L)]
        win = written & (entw < sent)
        wi = win.astype(jnp.int32)
        pos = cnt + plsc.cumsum(wi) - wi
        svec = base + k * _L + iota
        plsc.store_scatter(slist_v, [pos], svec, mask=win)
        plsc.store_scatter(jlist_v, [pos], wj, mask=win)
        return cnt + jnp.sum(wi)

    cnt = lax.fori_loop(0, _WCH, win_body, jnp.int32(0))

    stores[-1].wait()

    nch = (cnt + _L - 1) // _L

    def dma_body(c, carry):
        lanemask = (c * _L + iota) < cnt
        sv = slist_v[pl.ds(c * _L, _L)]
        jv = jlist_v[pl.ds(c * _L, _L)]
        sv = jnp.where(lanemask, sv, _gather16(sv, zero16))
        jv = jnp.where(lanemask, jv, _gather16(jv, zero16))
        pltpu.async_copy(valn_hbm.at[jv], rows_v, dma_sem).wait()
        pltpu.async_copy(rows_v, out_hbm.at[sv], dma_sem).wait()
        return carry

    lax.fori_loop(0, nch, dma_body, 0)


_sc_call = pl.kernel(
    _sc_body,
    out_type=jax.ShapeDtypeStruct((_M, _D), jnp.float32),
    mesh=plsc.VectorSubcoreMesh(
        core_axis_name="c", subcore_axis_name="s", num_cores=_NC,
        num_subcores=_NS,
    ),
    scratch_types=[
        pltpu.VMEM((_B,), jnp.int32),
        pltpu.VMEM((_B,), jnp.float32),
        pltpu.VMEM((_SPT,), jnp.float32),
        pltpu.VMEM((_SPT,), jnp.int32),
        pltpu.VMEM((_SPT,), jnp.int32),
        pltpu.VMEM((_SPT,), jnp.int32),
        pltpu.VMEM((_L, _D), jnp.float32),
        pltpu.VMEM((2, _RCH, _D), jnp.float32),
        pltpu.SemaphoreType.DMA,
        pltpu.SemaphoreType.DMA,
    ],
    compiler_params=pltpu.CompilerParams(needs_layout_passes=False),
)


def kernel(mem, ent_mem, val, prob, idx):
    ent3, val_n = _stats_call(prob.T, val)
    ent = ent3.reshape(_B)
    return _sc_call(mem, ent_mem, idx.astype(jnp.int32), ent, val_n)


# speedup vs baseline: 2.3055x; 1.0266x over previous
import functools

import jax
import jax.numpy as jnp
from jax import lax
from jax.experimental import pallas as pl
from jax.experimental.pallas import tpu as pltpu
from jax.experimental.pallas import tpu_sc as plsc

_BETA = 5.5
_M, _D, _B, _C = 65536, 512, 16384, 1000
_L = 16
_NC, _NS = 2, 16
_NW = _NC * _NS
_SPT = _M // _NW
_WCH = _SPT // _L
_SCH = _B // _L
_JBITS = 14
_ROWS_BLOCK = 256
_I32MAX = jnp.iinfo(jnp.int32).max


def _stats_body(probt_ref, val_ref, ent_ref, valn_ref):
    x = probt_ref[...] * _BETA
    m = jnp.max(x, axis=0, keepdims=True)
    xt = x - m
    e = jnp.exp(xt)
    z = jnp.sum(e, axis=0, keepdims=True)
    s = jnp.sum(e * xt, axis=0, keepdims=True)
    ent_ref[0, 0, :] = (jnp.log(z) - s / z)[0]
    v = val_ref[...]
    n = jnp.sqrt(jnp.sum(v * v, axis=-1, keepdims=True))
    valn_ref[...] = v / n


_NB = _B // _ROWS_BLOCK

_stats_call = pl.pallas_call(
    _stats_body,
    grid=(_NB,),
    in_specs=[
        pl.BlockSpec((_C, _ROWS_BLOCK), lambda i: (0, i)),
        pl.BlockSpec((_ROWS_BLOCK, _D), lambda i: (i, 0)),
    ],
    out_specs=[
        pl.BlockSpec((1, 1, _ROWS_BLOCK), lambda i: (i, 0, 0)),
        pl.BlockSpec((_ROWS_BLOCK, _D), lambda i: (i, 0)),
    ],
    out_shape=[
        jax.ShapeDtypeStruct((_NB, 1, _ROWS_BLOCK), jnp.float32),
        jax.ShapeDtypeStruct((_B, _D), jnp.float32),
    ],
)


def _gather16(x, idx):
    return lax.gather(
        x,
        idx[:, None],
        lax.GatherDimensionNumbers(
            offset_dims=(), collapsed_slice_dims=(0,), start_index_map=(0,)
        ),
        slice_sizes=(1,),
        mode=lax.GatherScatterMode.PROMISE_IN_BOUNDS,
    )


_RCH = 64
_NCP = _SPT // _RCH


def _sc_body(mem_hbm, entmem_hbm, idx_hbm, ent_hbm, valn_hbm, out_hbm,
             idx_v, ent_v, entmem_v, lastw_v, slist_v, jlist_v, rows_v,
             cbuf_v, copy_sem, dma_sem):
    wid = lax.axis_index("s") * _NC + lax.axis_index("c")
    base = wid * _SPT

    pltpu.sync_copy(idx_hbm, idx_v)
    pltpu.sync_copy(ent_hbm, ent_v)
    pltpu.sync_copy(entmem_hbm.at[pl.ds(base, _SPT)], entmem_v)

    iota = lax.iota(jnp.int32, _L)
    neg1 = jnp.full((_L,), -1, jnp.int32)
    big = jnp.full((_L,), _I32MAX, jnp.int32)
    nxt = jnp.minimum(iota + 1, _L - 1)
    zero16 = jnp.zeros((_L,), jnp.int32)

    def init_body(k, c):
        lastw_v[pl.ds(k * _L, _L)] = neg1
        return c

    lax.fori_loop(0, _WCH, init_body, 0)

    def scan_body(i, c):
        sv = idx_v[pl.ds(i * _L, _L)]
        jv = i * _L + iota
        inr = (sv >= base) & (sv < base + _SPT)
        comb = jnp.where(inr, (sv << _JBITS) + jv, big)
        srt = plsc.sort_key_val(comb, comb)[0]
        s_s = lax.shift_right_arithmetic(srt, _JBITS)
        j_s = lax.bitwise_and(srt, _B - 1)
        valid = srt != big
        s_nxt = _gather16(s_s, nxt)
        lastrun = (s_s != s_nxt) | (iota == _L - 1)
        plsc.store_scatter(lastw_v, [s_s - base], j_s, mask=valid & lastrun)
        return c

    _SEG = _SCH // _NCP
    stores = [None] * _NCP
    gathers = [None] * _NCP
    gathers[0] = pltpu.async_copy(
        mem_hbm.at[pl.ds(base, _RCH)], cbuf_v.at[0], dma_sem
    )
    for c in range(_NCP):
        gathers[c].wait()
        stores[c] = pltpu.async_copy(
            cbuf_v.at[c % 2], out_hbm.at[pl.ds(base + c * _RCH, _RCH)],
            copy_sem,
        )
        if c + 1 < _NCP:
            if c >= 1:
                stores[c - 1].wait()
            gathers[c + 1] = pltpu.async_copy(
                mem_hbm.at[pl.ds(base + (c + 1) * _RCH, _RCH)],
                cbuf_v.at[(c + 1) % 2], dma_sem,
            )
        lax.fori_loop(c * _SEG, (c + 1) * _SEG, scan_body, 0)

    def win_body(k, cnt):
        wj = lastw_v[pl.ds(k * _L, _L)]
        written = wj >= 0
        entw = plsc.load_gather(ent_v, [wj], mask=written)
        sent = entmem_v[pl.ds(k * _L, _L)]
        win = written & (entw < sent)
        wi = win.astype(jnp.int32)
        pos = cnt + plsc.cumsum(wi) - wi
        svec = base + k * _L + iota
        plsc.store_scatter(slist_v, [pos], svec, mask=win)
        plsc.store_scatter(jlist_v, [pos], wj, mask=win)
        return cnt + jnp.sum(wi)

    cnt = lax.fori_loop(0, _WCH, win_body, jnp.int32(0))

    stores[-1].wait()

    nch = (cnt + _L - 1) // _L

    def _drain_one_scatter():
        pltpu.make_async_copy(
            valn_hbm.at[zero16], rows_v.at[0], copy_sem
        ).wait()

    def dma_body(c, carry):
        b = lax.bitwise_and(c, 1)
        lanemask = (c * _L + iota) < cnt
        sv = slist_v[pl.ds(c * _L, _L)]
        jv = jlist_v[pl.ds(c * _L, _L)]
        sv = jnp.where(lanemask, sv, _gather16(sv, zero16))
        jv = jnp.where(lanemask, jv, _gather16(jv, zero16))

        @pl.when(c >= 2)
        def _():
            _drain_one_scatter()

        pltpu.async_copy(valn_hbm.at[jv], rows_v.at[b], dma_sem).wait()
        pltpu.async_copy(rows_v.at[b], out_hbm.at[sv], copy_sem)
        return carry

    lax.fori_loop(0, nch, dma_body, 0)

    @pl.when(nch >= 1)
    def _():
        _drain_one_scatter()

    @pl.when(nch >= 2)
    def _():
        _drain_one_scatter()


_sc_call = pl.kernel(
    _sc_body,
    out_type=jax.ShapeDtypeStruct((_M, _D), jnp.float32),
    mesh=plsc.VectorSubcoreMesh(
        core_axis_name="c", subcore_axis_name="s", num_cores=_NC,
        num_subcores=_NS,
    ),
    scratch_types=[
        pltpu.VMEM((_B,), jnp.int32),
        pltpu.VMEM((_B,), jnp.float32),
        pltpu.VMEM((_SPT,), jnp.float32),
        pltpu.VMEM((_SPT,), jnp.int32),
        pltpu.VMEM((_SPT,), jnp.int32),
        pltpu.VMEM((_SPT,), jnp.int32),
        pltpu.VMEM((2, _L, _D), jnp.float32),
        pltpu.VMEM((2, _RCH, _D), jnp.float32),
        pltpu.SemaphoreType.DMA,
        pltpu.SemaphoreType.DMA,
    ],
    compiler_params=pltpu.CompilerParams(needs_layout_passes=False),
)


def kernel(mem, ent_mem, val, prob, idx):
    ent3, val_n = _stats_call(prob.T, val)
    ent = ent3.reshape(_B)
    return _sc_call(mem, ent_mem, idx.astype(jnp.int32), ent, val_n)


# speedup vs baseline: 2.3089x; 1.0015x over previous
import jax
import jax.numpy as jnp
from jax import lax
from jax.experimental import pallas as pl
from jax.experimental.pallas import tpu as pltpu
from jax.experimental.pallas import tpu_sc as plsc

_BETA = 5.5
_M, _D, _B, _C = 65536, 512, 16384, 1000
_L = 16
_NC, _NS = 2, 16
_NW = _NC * _NS
_SPT = _M // _NW
_WCH = _SPT // _L
_SCH = _B // _L
_JBITS = 14
_ROWS_BLOCK = 256
_I32MAX = jnp.iinfo(jnp.int32).max


def _stats_body(probt_ref, val_ref, ent_ref, valn_ref):
    x = probt_ref[...] * _BETA
    m = jnp.max(x, axis=0, keepdims=True)
    xt = x - m
    e = jnp.exp(xt)
    z = jnp.sum(e, axis=0, keepdims=True)
    s = jnp.sum(e * xt, axis=0, keepdims=True)
    ent_ref[0, 0, :] = (jnp.log(z) - s / z)[0]
    v = val_ref[...]
    n = jnp.sqrt(jnp.sum(v * v, axis=-1, keepdims=True))
    valn_ref[...] = v / n


_NB = _B // _ROWS_BLOCK

_stats_call = pl.pallas_call(
    _stats_body,
    grid=(_NB,),
    in_specs=[
        pl.BlockSpec((_C, _ROWS_BLOCK), lambda i: (0, i)),
        pl.BlockSpec((_ROWS_BLOCK, _D), lambda i: (i, 0)),
    ],
    out_specs=[
        pl.BlockSpec((1, 1, _ROWS_BLOCK), lambda i: (i, 0, 0)),
        pl.BlockSpec((_ROWS_BLOCK, _D), lambda i: (i, 0)),
    ],
    out_shape=[
        jax.ShapeDtypeStruct((_NB, 1, _ROWS_BLOCK), jnp.float32),
        jax.ShapeDtypeStruct((_B, _D), jnp.float32),
    ],
)


def _gather16(x, idx):
    return lax.gather(
        x,
        idx[:, None],
        lax.GatherDimensionNumbers(
            offset_dims=(), collapsed_slice_dims=(0,), start_index_map=(0,)
        ),
        slice_sizes=(1,),
        mode=lax.GatherScatterMode.PROMISE_IN_BOUNDS,
    )


_RCH = 64
_NCP = _SPT // _RCH


def _sc_body(mem_hbm, entmem_hbm, idx_hbm, ent_hbm, valn_hbm, out_hbm,
             idx_v, ent_v, entmem_v, lastw_v, slist_v, jlist_v, rows_v,
             cbuf_v, copy_sem, dma_sem):
    wid = lax.axis_index("s") * _NC + lax.axis_index("c")
    base = wid * _SPT

    pltpu.sync_copy(idx_hbm, idx_v)
    pltpu.sync_copy(ent_hbm, ent_v)
    pltpu.sync_copy(entmem_hbm.at[pl.ds(base, _SPT)], entmem_v)

    iota = lax.iota(jnp.int32, _L)
    neg1 = jnp.full((_L,), -1, jnp.int32)
    big = jnp.full((_L,), _I32MAX, jnp.int32)
    nxt = jnp.minimum(iota + 1, _L - 1)
    zero16 = jnp.zeros((_L,), jnp.int32)

    def init_body(k, c):
        lastw_v[pl.ds(k * _L, _L)] = neg1
        return c

    lax.fori_loop(0, _WCH, init_body, 0)

    def scan_body(i, c):
        sv = idx_v[pl.ds(i * _L, _L)]
        jv = i * _L + iota
        inr = (sv >= base) & (sv < base + _SPT)
        comb = jnp.where(inr, (sv << _JBITS) + jv, big)
        srt = plsc.sort_key_val(comb, comb)[0]
        s_s = lax.shift_right_arithmetic(srt, _JBITS)
        j_s = lax.bitwise_and(srt, _B - 1)
        valid = srt != big
        s_nxt = _gather16(s_s, nxt)
        lastrun = (s_s != s_nxt) | (iota == _L - 1)
        plsc.store_scatter(lastw_v, [s_s - base], j_s, mask=valid & lastrun)
        return c

    _SEG = _SCH // _NCP
    stores = [None] * _NCP
    gathers = [None] * _NCP
    gathers[0] = pltpu.async_copy(
        mem_hbm.at[pl.ds(base, _RCH)], cbuf_v.at[0], dma_sem
    )
    for c in range(_NCP):
        gathers[c].wait()
        stores[c] = pltpu.async_copy(
            cbuf_v.at[c % 2], out_hbm.at[pl.ds(base + c * _RCH, _RCH)],
            copy_sem,
        )
        if c + 1 < _NCP:
            if c >= 1:
                stores[c - 1].wait()
            gathers[c + 1] = pltpu.async_copy(
                mem_hbm.at[pl.ds(base + (c + 1) * _RCH, _RCH)],
                cbuf_v.at[(c + 1) % 2], dma_sem,
            )
        lax.fori_loop(c * _SEG, (c + 1) * _SEG, scan_body, 0)

    def win_body(k, cnt):
        wj = lastw_v[pl.ds(k * _L, _L)]
        written = wj >= 0
        entw = plsc.load_gather(ent_v, [wj], mask=written)
        sent = entmem_v[pl.ds(k * _L, _L)]
        win = written & (entw < sent)
        wi = win.astype(jnp.int32)
        pos = cnt + plsc.cumsum(wi) - wi
        svec = base + k * _L + iota
        plsc.store_scatter(slist_v, [pos], svec, mask=win)
        plsc.store_scatter(jlist_v, [pos], wj, mask=win)
        return cnt + jnp.sum(wi)

    cnt = lax.fori_loop(0, _WCH, win_body, jnp.int32(0))

    stores[-1].wait()

    nch = (cnt + _L - 1) // _L

    def _drain_one_scatter():
        pltpu.make_async_copy(
            valn_hbm.at[zero16], rows_v.at[0], copy_sem
        ).wait()

    def dma_body(c, carry):
        b = lax.bitwise_and(c, 1)
        lanemask = (c * _L + iota) < cnt
        sv = slist_v[pl.ds(c * _L, _L)]
        jv = jlist_v[pl.ds(c * _L, _L)]
        sv = jnp.where(lanemask, sv, _gather16(sv, zero16))
        jv = jnp.where(lanemask, jv, _gather16(jv, zero16))

        @pl.when(c >= 2)
        def _():
            _drain_one_scatter()

        pltpu.async_copy(valn_hbm.at[jv], rows_v.at[b], dma_sem).wait()
        pltpu.async_copy(rows_v.at[b], out_hbm.at[sv], copy_sem)
        return carry

    lax.fori_loop(0, nch, dma_body, 0)

    @pl.when(nch >= 1)
    def _():
        _drain_one_scatter()

    @pl.when(nch >= 2)
    def _():
        _drain_one_scatter()


_sc_call = pl.kernel(
    _sc_body,
    out_type=jax.ShapeDtypeStruct((_M, _D), jnp.float32),
    mesh=plsc.VectorSubcoreMesh(
        core_axis_name="c", subcore_axis_name="s", num_cores=_NC,
        num_subcores=_NS,
    ),
    scratch_types=[
        pltpu.VMEM((_B,), jnp.int32),
        pltpu.VMEM((_B,), jnp.float32),
        pltpu.VMEM((_SPT,), jnp.float32),
        pltpu.VMEM((_SPT,), jnp.int32),
        pltpu.VMEM((_SPT,), jnp.int32),
        pltpu.VMEM((_SPT,), jnp.int32),
        pltpu.VMEM((2, _L, _D), jnp.float32),
        pltpu.VMEM((2, _RCH, _D), jnp.float32),
        pltpu.SemaphoreType.DMA,
        pltpu.SemaphoreType.DMA,
    ],
    compiler_params=pltpu.CompilerParams(needs_layout_passes=False),
)


def kernel(mem, ent_mem, val, prob, idx):
    ent3, val_n = _stats_call(prob.T, val)
    ent = ent3.reshape(_B)
    return _sc_call(mem, ent_mem, idx.astype(jnp.int32), ent, val_n)
